# baseline (speedup 1.0000x reference)
import functools

import jax
import jax.numpy as jnp
from jax import lax
from jax.experimental import pallas as pl
from jax.experimental.pallas import tpu as pltpu
from jax.experimental.pallas import tpu_sc as plsc

_N = 10000
_E = 160000
_D_IN = 256
_D_OUT = 512

_H = 128
_CW = 8
_NSUB = 16
_NCORE = 2
_EDGES_PER_SUB = _E // _NSUB
_CHUNK = 80
_NCHUNK = _EDGES_PER_SUB // _CHUNK
_NPAD = 10240
_ROWS_PER_SUB = _NPAD // _NSUB


def _sc_aggregate(x2, edge3, zrows, zcnt, ones_rows):
    mesh = plsc.VectorSubcoreMesh(core_axis_name="c", subcore_axis_name="s")

    @functools.partial(
        pl.kernel,
        mesh=mesh,
        compiler_params=pltpu.CompilerParams(use_tc_tiling_on_sc=False),
        out_type=(
            jax.ShapeDtypeStruct((_NCORE, _NPAD, _H), jnp.float32),
            jax.ShapeDtypeStruct((_NPAD, _CW), jnp.float32),
        ),
        scratch_types=[
            pltpu.VMEM((_NCHUNK, _CHUNK), jnp.int32),
            pltpu.VMEM((_NCHUNK, _CHUNK), jnp.int32),
            pltpu.VMEM((_CHUNK, _H), jnp.float32),
            pltpu.VMEM((_CHUNK, _H), jnp.float32),
            pltpu.VMEM((_CHUNK, _CW), jnp.float32),
            pltpu.VMEM_SHARED((_NPAD, _H), jnp.float32),
            pltpu.VMEM_SHARED((_NPAD, _CW), jnp.float32),
            pltpu.SemaphoreType.DMA,
            pltpu.SemaphoreType.DMA,
            pltpu.SemaphoreType.DMA,
            pltpu.SemaphoreType.DMA,
            pltpu.SemaphoreType.DMA,
            pltpu.SemaphoreType.DMA,
        ],
    )
    def k(x2_hbm, edge_hbm, z_hbm, zc_hbm, ones_hbm,
          feat_out, cnt_out,
          src_v, dst_v, rows0_v, rows1_v, ones_v, accf, accc,
          rsem0, rsem1, csem0, csem1, ssem0, ssem1):
        c = lax.axis_index("c")
        s = lax.axis_index("s")
        row0 = s * _ROWS_PER_SUB

        rows = (rows0_v, rows1_v)
        rsem = (rsem0, rsem1)
        csem = (csem0, csem1)
        ssem = (ssem0, ssem1)

        pltpu.sync_copy(z_hbm, accf.at[pl.ds(row0, _ROWS_PER_SUB)])
        pltpu.sync_copy(zc_hbm, accc.at[pl.ds(row0, _ROWS_PER_SUB)])
        pltpu.sync_copy(edge_hbm.at[0, s], src_v)
        pltpu.sync_copy(edge_hbm.at[1, s], dst_v)
        pltpu.sync_copy(ones_hbm, ones_v)

        def xform(j, carry):
            for t in range(_CHUNK // 16):
                v = src_v[j, pl.ds(t * 16, 16)]
                src_v[j, pl.ds(t * 16, 16)] = v + v + c
            return carry

        lax.fori_loop(0, _NCHUNK, xform, 0)
        plsc.subcore_barrier()

        def gather(j, p):
            pltpu.async_copy(x2_hbm.at[src_v.at[j]], rows[p], rsem[p])

        def wait_rows(p):
            pltpu.make_async_copy(x2_hbm.at[pl.ds(0, _CHUNK)], rows[p],
                                  rsem[p]).wait()

        def wait_scat(p):
            pltpu.make_async_copy(x2_hbm.at[pl.ds(0, _CHUNK)], rows[p],
                                  ssem[p]).wait()

        def wait_cnt(p):
            pltpu.make_async_copy(ones_hbm, ones_v, csem[p]).wait()

        gather(0, 0)

        def step(j, p):
            @pl.when(j + 1 < _NCHUNK)
            def _():
                @pl.when(j >= 1)
                def _():
                    wait_scat(1 - p)

                gather(j + 1, 1 - p)

            @pl.when(j >= 2)
            def _():
                wait_cnt(p)

            pltpu.async_copy(ones_v, accc.at[dst_v.at[j]], csem[p], add=True)

            wait_rows(p)
            pltpu.async_copy(rows[p], accf.at[dst_v.at[j]], ssem[p], add=True)

        def body(j, carry):
            @pl.when(j % 2 == 0)
            def _():
                step(j, 0)

            @pl.when(j % 2 == 1)
            def _():
                step(j, 1)

            return carry

        lax.fori_loop(0, _NCHUNK, body, 0)
        wait_scat(0)
        wait_scat(1)
        wait_cnt(0)
        wait_cnt(1)
        plsc.subcore_barrier()

        pltpu.sync_copy(accf.at[pl.ds(row0, _ROWS_PER_SUB)],
                        feat_out.at[c, pl.ds(row0, _ROWS_PER_SUB)])

        @pl.when(c == 0)
        def _():
            pltpu.sync_copy(accc.at[pl.ds(row0, _ROWS_PER_SUB)],
                            cnt_out.at[pl.ds(row0, _ROWS_PER_SUB)])

    return k(x2, edge3, zrows, zcnt, ones_rows)


_TC_ROWS = 1000


def _tc_linear(feats, cnt, x, wl0, wl1, wr, b):
    def body(a0_ref, a1_ref, c_ref, x_ref, wl0_ref, wl1_ref, wr_ref, b_ref,
             o_ref):
        denom = jnp.maximum(c_ref[:, 0:1], 1.0)
        n0 = a0_ref[0] / denom
        n1 = a1_ref[0] / denom
        dims = (((1,), (1,)), ((), ()))
        out = (lax.dot_general(n0, wl0_ref[...], dims,
                               preferred_element_type=jnp.float32)
               + lax.dot_general(n1, wl1_ref[...], dims,
                                 preferred_element_type=jnp.float32)
               + lax.dot_general(x_ref[...], wr_ref[...], dims,
                                 preferred_element_type=jnp.float32)
               + b_ref[...])
        o_ref[...] = jnp.maximum(out, 0.0)

    return pl.pallas_call(
        body,
        grid=(_N // _TC_ROWS,),
        in_specs=[
            pl.BlockSpec((1, _TC_ROWS, _H), lambda i: (0, i, 0)),
            pl.BlockSpec((1, _TC_ROWS, _H), lambda i: (1, i, 0)),
            pl.BlockSpec((_TC_ROWS, _CW), lambda i: (i, 0)),
            pl.BlockSpec((_TC_ROWS, _D_IN), lambda i: (i, 0)),
            pl.BlockSpec((_D_OUT, _H), lambda i: (0, 0)),
            pl.BlockSpec((_D_OUT, _H), lambda i: (0, 0)),
            pl.BlockSpec((_D_OUT, _D_IN), lambda i: (0, 0)),
            pl.BlockSpec((1, _D_OUT), lambda i: (0, 0)),
        ],
        out_specs=pl.BlockSpec((_TC_ROWS, _D_OUT), lambda i: (i, 0)),
        out_shape=jax.ShapeDtypeStruct((_N, _D_OUT), jnp.float32),
    )(feats, feats, cnt, x, wl0, wl1, wr, b)


def kernel(x, edge_index, W_l, b_l, W_r):
    x2 = x.reshape(_NCORE * _N, _H)
    edge3 = edge_index.reshape(2, _NSUB, _NCHUNK, _CHUNK)
    zrows = jnp.zeros((_ROWS_PER_SUB, _H), jnp.float32)
    zcnt = jnp.zeros((_ROWS_PER_SUB, _CW), jnp.float32)
    ones_rows = jnp.concatenate(
        [jnp.ones((_CHUNK, 1), jnp.float32),
         jnp.zeros((_CHUNK, _CW - 1), jnp.float32)], axis=1)

    feats, cnt = _sc_aggregate(x2, edge3, zrows, zcnt, ones_rows)

    return _tc_linear(feats, cnt, x, W_l[:, :_H], W_l[:, _H:], W_r,
                      b_l.reshape(1, _D_OUT))
